# Initial kernel scaffold; baseline (speedup 1.0000x reference)
import jax
import jax.numpy as jnp
from jax.experimental import pallas as pl


def kernel(x, W, b):
    raise NotImplementedError("write your pallas kernel here")


# speedup vs baseline: 3.1592x; 3.1592x over previous
import functools

import jax
import jax.numpy as jnp
from jax import lax
from jax.experimental import pallas as pl
from jax.experimental.pallas import tpu as pltpu, tpu_sc as plsc

_N_TOKENS = 32768
_D_MODEL = 768
_N_EXPERTS = 64
_LANES = 16

_MM_BLK = 1024

_NW = 32
_ROWS_PER_W = _N_TOKENS // _NW
_CHUNK = 512
_NCHUNK = _ROWS_PER_W // _CHUNK


def _matmul_body(x_ref, w_ref, b_ref, o_ref):
    acc = lax.dot_general(
        w_ref[...], x_ref[...],
        dimension_numbers=(((1,), (1,)), ((), ())),
        preferred_element_type=jnp.float32,
    )
    o_ref[...] = acc + b_ref[:, 0:1]


def _logits_t(x, W, b2d):
    return pl.pallas_call(
        _matmul_body,
        grid=(_N_TOKENS // _MM_BLK,),
        in_specs=[
            pl.BlockSpec((_MM_BLK, _D_MODEL), lambda i: (i, 0)),
            pl.BlockSpec((_N_EXPERTS, _D_MODEL), lambda i: (0, 0)),
            pl.BlockSpec((_N_EXPERTS, 128), lambda i: (0, 0)),
        ],
        out_specs=pl.BlockSpec((_N_EXPERTS, _MM_BLK), lambda i: (0, i)),
        out_shape=jax.ShapeDtypeStruct((_N_EXPERTS, _N_TOKENS), jnp.float32),
    )(x, W, b2d)


def _route_body(lt_hbm, gate_hbm, idx_hbm, lt_v, out_v, idx_v):
    wid = lax.axis_index("s") * 2 + lax.axis_index("c")
    zeros_f = jnp.zeros((_LANES,), jnp.float32)
    lane = lax.iota(jnp.int32, _LANES)

    for c in range(_NCHUNK):
        base = wid * _ROWS_PER_W + c * _CHUNK

        pltpu.sync_copy(lt_hbm.at[:, pl.ds(base, _CHUNK)], lt_v)

        def zero_body(r, _):
            out_v[pl.ds(r * _LANES, _LANES)] = zeros_f
            return 0

        lax.fori_loop(0, _CHUNK * _N_EXPERTS // _LANES, zero_body, 0)

        def group_body(g, _):
            def expert_body(e, carry):
                m0, i0, m1, i1 = carry
                v = lt_v[e, pl.ds(g * _LANES, _LANES)]
                ev = jnp.full((_LANES,), 0, jnp.int32) + e
                gt0 = v > m0
                gt1 = v > m1
                i1n = jnp.where(gt0, i0, jnp.where(gt1, ev, i1))
                m1n = jnp.where(gt0, m0, jnp.where(gt1, v, m1))
                i0n = jnp.where(gt0, ev, i0)
                m0n = jnp.where(gt0, v, m0)
                return (m0n, i0n, m1n, i1n)

            init = (
                jnp.full((_LANES,), -jnp.inf, jnp.float32),
                jnp.zeros((_LANES,), jnp.int32),
                jnp.full((_LANES,), -jnp.inf, jnp.float32),
                jnp.zeros((_LANES,), jnp.int32),
            )
            m0, i0, m1, i1 = lax.fori_loop(0, _N_EXPERTS, expert_body, init)

            d = jnp.exp(m1 - m0)
            s = d + 1.0
            p0 = 1.0 / s
            p1 = d / s

            rows = g * _LANES + lane
            flat = rows * _N_EXPERTS
            plsc.store_scatter(out_v, [flat + i0], p0)
            plsc.store_scatter(out_v, [flat + i1], p1)
            rows2 = rows * 2
            plsc.store_scatter(idx_v, [rows2], i0)
            plsc.store_scatter(idx_v, [rows2 + 1], i1)
            return 0

        lax.fori_loop(0, _CHUNK // _LANES, group_body, 0)

        pltpu.sync_copy(out_v, gate_hbm.at[pl.ds(base * _N_EXPERTS, _CHUNK * _N_EXPERTS)])
        pltpu.sync_copy(idx_v, idx_hbm.at[pl.ds(base * 2, _CHUNK * 2)])


_route = functools.partial(
    pl.kernel,
    mesh=plsc.VectorSubcoreMesh(core_axis_name="c", subcore_axis_name="s"),
    out_type=(
        jax.ShapeDtypeStruct((_N_TOKENS * _N_EXPERTS,), jnp.float32),
        jax.ShapeDtypeStruct((_N_TOKENS * 2,), jnp.int32),
    ),
    scratch_types=[
        pltpu.VMEM((_N_EXPERTS, _CHUNK), jnp.float32),
        pltpu.VMEM((_CHUNK * _N_EXPERTS,), jnp.float32),
        pltpu.VMEM((_CHUNK * 2,), jnp.int32),
    ],
    compiler_params=pltpu.CompilerParams(needs_layout_passes=False),
)(_route_body)


@jax.jit
def kernel(x, W, b):
    b2d = jnp.broadcast_to(b[:, None], (_N_EXPERTS, 128))
    lt = _logits_t(x, W, b2d)
    gate_flat, idx_flat = _route(lt)
    return (
        gate_flat.reshape(_N_TOKENS, _N_EXPERTS),
        idx_flat.reshape(_N_TOKENS, 2),
    )


# speedup vs baseline: 3.7467x; 1.1859x over previous
import functools

import jax
import jax.numpy as jnp
from jax import lax
from jax.experimental import pallas as pl
from jax.experimental.pallas import tpu as pltpu, tpu_sc as plsc

_N_TOKENS = 32768
_D_MODEL = 768
_N_EXPERTS = 64
_LANES = 16

_MM_BLK = 1024

_NW = 32
_ROWS_PER_W = _N_TOKENS // _NW
_CHUNK = 256
_NCHUNK = _ROWS_PER_W // _CHUNK
_GROUPS = _CHUNK // _LANES


def _matmul_body(x_ref, w_ref, b_ref, o_ref):
    acc = lax.dot_general(
        w_ref[...], x_ref[...],
        dimension_numbers=(((1,), (1,)), ((), ())),
        preferred_element_type=jnp.float32,
    )
    o_ref[...] = acc + b_ref[:, 0:1]


def _logits_t(x, W, b2d):
    return pl.pallas_call(
        _matmul_body,
        grid=(_N_TOKENS // _MM_BLK,),
        in_specs=[
            pl.BlockSpec((_MM_BLK, _D_MODEL), lambda i: (i, 0)),
            pl.BlockSpec((_N_EXPERTS, _D_MODEL), lambda i: (0, 0)),
            pl.BlockSpec((_N_EXPERTS, 128), lambda i: (0, 0)),
        ],
        out_specs=pl.BlockSpec((_N_EXPERTS, _MM_BLK), lambda i: (0, i)),
        out_shape=jax.ShapeDtypeStruct((_N_EXPERTS, _N_TOKENS), jnp.float32),
    )(x, W, b2d)


def _route_body(lt_hbm, gate_hbm, idx_hbm,
                lt_v0, lt_v1, out_v0, out_v1, idx_v0, idx_v1, pos_v0, pos_v1,
                lt_s0, lt_s1, out_s0, out_s1):
    wid = lax.axis_index("s") * 2 + lax.axis_index("c")
    tok0 = wid * _ROWS_PER_W
    zeros_f = jnp.zeros((_LANES,), jnp.float32)
    lane = lax.iota(jnp.int32, _LANES)
    lt_vs = (lt_v0, lt_v1)
    out_vs = (out_v0, out_v1)
    idx_vs = (idx_v0, idx_v1)
    pos_vs = (pos_v0, pos_v1)
    lt_sems = (lt_s0, lt_s1)
    out_sems = (out_s0, out_s1)

    lt_dma = [None] * _NCHUNK
    lt_dma[0] = pltpu.async_copy(
        lt_hbm.at[:, pl.ds(tok0, _CHUNK)], lt_v0, lt_s0)

    def zero_body(r, _):
        out_v0[pl.ds(r * _LANES, _LANES)] = zeros_f
        out_v1[pl.ds(r * _LANES, _LANES)] = zeros_f
        return 0

    lax.fori_loop(0, _CHUNK * _N_EXPERTS // _LANES, zero_body, 0)

    out_dma = [None] * _NCHUNK
    idx_dma = [None] * _NCHUNK

    for c in range(_NCHUNK):
        p = c & 1
        lt_v, out_v, idx_v, pos_v = lt_vs[p], out_vs[p], idx_vs[p], pos_vs[p]
        base = tok0 + c * _CHUNK
        lt_dma[c].wait()
        if c + 1 < _NCHUNK:
            lt_dma[c + 1] = pltpu.async_copy(
                lt_hbm.at[:, pl.ds(base + _CHUNK, _CHUNK)],
                lt_vs[p ^ 1], lt_sems[p ^ 1])
        if c >= 2:
            out_dma[c - 2].wait()
            idx_dma[c - 2].wait()

            def rezero_body(j, _):
                pos = pos_v[pl.ds(j * _LANES, _LANES)]
                plsc.store_scatter(out_v, [pos], zeros_f)
                return 0

            lax.fori_loop(0, 2 * _GROUPS, rezero_body, 0)

        def group_body(g, _):
            m0 = jnp.full((_LANES,), -jnp.inf, jnp.float32)
            m1 = jnp.full((_LANES,), -jnp.inf, jnp.float32)
            i0 = jnp.zeros((_LANES,), jnp.int32)
            i1 = jnp.zeros((_LANES,), jnp.int32)
            off = g * _LANES
            for e in range(_N_EXPERTS):
                v = lt_v[e, pl.ds(off, _LANES)]
                ev = jnp.full((_LANES,), e, jnp.int32)
                gt0 = v > m0
                gt1 = v > m1
                i1 = jnp.where(gt0, i0, jnp.where(gt1, ev, i1))
                m1 = jnp.where(gt0, m0, jnp.where(gt1, v, m1))
                i0 = jnp.where(gt0, ev, i0)
                m0 = jnp.where(gt0, v, m0)

            d = jnp.exp(m1 - m0)
            s = d + 1.0
            p0 = 1.0 / s
            p1 = d / s

            rows = off + lane
            f0 = rows * _N_EXPERTS + i0
            f1 = rows * _N_EXPERTS + i1
            plsc.store_scatter(out_v, [f0], p0)
            plsc.store_scatter(out_v, [f1], p1)
            pos_v[pl.ds(2 * off, _LANES)] = f0
            pos_v[pl.ds(2 * off + _LANES, _LANES)] = f1
            rows2 = rows * 2
            plsc.store_scatter(idx_v, [rows2], i0)
            plsc.store_scatter(idx_v, [rows2 + 1], i1)
            return 0

        lax.fori_loop(0, _GROUPS, group_body, 0)

        out_dma[c] = pltpu.async_copy(
            out_v,
            gate_hbm.at[pl.ds(base * _N_EXPERTS, _CHUNK * _N_EXPERTS)],
            out_sems[p])
        idx_dma[c] = pltpu.async_copy(
            idx_v, idx_hbm.at[pl.ds(base * 2, _CHUNK * 2)],
            out_sems[p])

    out_dma[_NCHUNK - 2].wait()
    idx_dma[_NCHUNK - 2].wait()
    out_dma[_NCHUNK - 1].wait()
    idx_dma[_NCHUNK - 1].wait()


_route = functools.partial(
    pl.kernel,
    mesh=plsc.VectorSubcoreMesh(core_axis_name="c", subcore_axis_name="s"),
    out_type=(
        jax.ShapeDtypeStruct((_N_TOKENS * _N_EXPERTS,), jnp.float32),
        jax.ShapeDtypeStruct((_N_TOKENS * 2,), jnp.int32),
    ),
    scratch_types=[
        pltpu.VMEM((_N_EXPERTS, _CHUNK), jnp.float32),
        pltpu.VMEM((_N_EXPERTS, _CHUNK), jnp.float32),
        pltpu.VMEM((_CHUNK * _N_EXPERTS,), jnp.float32),
        pltpu.VMEM((_CHUNK * _N_EXPERTS,), jnp.float32),
        pltpu.VMEM((_CHUNK * 2,), jnp.int32),
        pltpu.VMEM((_CHUNK * 2,), jnp.int32),
        pltpu.VMEM((_CHUNK * 2,), jnp.int32),
        pltpu.VMEM((_CHUNK * 2,), jnp.int32),
        pltpu.SemaphoreType.DMA,
        pltpu.SemaphoreType.DMA,
        pltpu.SemaphoreType.DMA,
        pltpu.SemaphoreType.DMA,
    ],
    compiler_params=pltpu.CompilerParams(needs_layout_passes=False),
)(_route_body)


@jax.jit
def kernel(x, W, b):
    b2d = jnp.broadcast_to(b[:, None], (_N_EXPERTS, 128))
    lt = _logits_t(x, W, b2d)
    gate_flat, idx_flat = _route(lt)
    return (
        gate_flat.reshape(_N_TOKENS, _N_EXPERTS),
        idx_flat.reshape(_N_TOKENS, 2),
    )


# speedup vs baseline: 4.0742x; 1.0874x over previous
import functools

import jax
import jax.numpy as jnp
from jax import lax
from jax.experimental import pallas as pl
from jax.experimental.pallas import tpu as pltpu, tpu_sc as plsc

_N_TOKENS = 32768
_D_MODEL = 768
_N_EXPERTS = 64
_LANES = 16

_MM_BLK = 1024

_NW = 32
_ROWS_PER_W = _N_TOKENS // _NW
_CHUNK = 256
_NCHUNK = _ROWS_PER_W // _CHUNK
_GROUPS = _CHUNK // _LANES

_OUT_BLK = 4096


def _matmul_body(x_ref, w_ref, b_ref, o_ref):
    acc = lax.dot_general(
        w_ref[...], x_ref[...],
        dimension_numbers=(((1,), (1,)), ((), ())),
        preferred_element_type=jnp.float32,
    )
    o_ref[...] = acc + b_ref[:, 0:1]


def _logits_t(x, W, b2d):
    return pl.pallas_call(
        _matmul_body,
        grid=(_N_TOKENS // _MM_BLK,),
        in_specs=[
            pl.BlockSpec((_MM_BLK, _D_MODEL), lambda i: (i, 0)),
            pl.BlockSpec((_N_EXPERTS, _D_MODEL), lambda i: (0, 0)),
            pl.BlockSpec((_N_EXPERTS, 128), lambda i: (0, 0)),
        ],
        out_specs=pl.BlockSpec((_N_EXPERTS, _MM_BLK), lambda i: (0, i)),
        out_shape=jax.ShapeDtypeStruct((_N_EXPERTS, _N_TOKENS), jnp.float32),
    )(x, W, b2d)


def _route_body(lt_hbm, p1_hbm, i0_hbm, i1_hbm,
                lt_v0, lt_v1, p1_v, i0_v, i1_v, lt_s0, lt_s1):
    wid = lax.axis_index("s") * 2 + lax.axis_index("c")
    tok0 = wid * _ROWS_PER_W
    lane = lax.iota(jnp.int32, _LANES)
    lt_vs = (lt_v0, lt_v1)
    lt_sems = (lt_s0, lt_s1)

    lt_dma = [None] * _NCHUNK
    lt_dma[0] = pltpu.async_copy(
        lt_hbm.at[:, pl.ds(tok0, _CHUNK)], lt_v0, lt_s0)

    for c in range(_NCHUNK):
        p = c & 1
        lt_v = lt_vs[p]
        base = tok0 + c * _CHUNK
        lt_dma[c].wait()
        if c + 1 < _NCHUNK:
            lt_dma[c + 1] = pltpu.async_copy(
                lt_hbm.at[:, pl.ds(base + _CHUNK, _CHUNK)],
                lt_vs[p ^ 1], lt_sems[p ^ 1])

        def group_body(g, _):
            m0 = jnp.full((_LANES,), -jnp.inf, jnp.float32)
            m1 = jnp.full((_LANES,), -jnp.inf, jnp.float32)
            i0 = jnp.zeros((_LANES,), jnp.int32)
            i1 = jnp.zeros((_LANES,), jnp.int32)
            off = g * _LANES
            for e in range(_N_EXPERTS):
                v = lt_v[e, pl.ds(off, _LANES)]
                ev = jnp.full((_LANES,), e, jnp.int32)
                gt0 = v > m0
                gt1 = v > m1
                i1 = jnp.where(gt0, i0, jnp.where(gt1, ev, i1))
                m1 = jnp.where(gt0, m0, jnp.where(gt1, v, m1))
                i0 = jnp.where(gt0, ev, i0)
                m0 = jnp.where(gt0, v, m0)

            d = jnp.exp(m1 - m0)
            p1 = d / (d + 1.0)

            coff = c * _CHUNK + off
            p1_v[pl.ds(coff, _LANES)] = p1
            i0_v[pl.ds(coff, _LANES)] = i0
            i1_v[pl.ds(coff, _LANES)] = i1
            return 0

        lax.fori_loop(0, _GROUPS, group_body, 0)

    pltpu.sync_copy(p1_v, p1_hbm.at[pl.ds(tok0, _ROWS_PER_W)])
    pltpu.sync_copy(i0_v, i0_hbm.at[pl.ds(tok0, _ROWS_PER_W)])
    pltpu.sync_copy(i1_v, i1_hbm.at[pl.ds(tok0, _ROWS_PER_W)])


_route = functools.partial(
    pl.kernel,
    mesh=plsc.VectorSubcoreMesh(core_axis_name="c", subcore_axis_name="s"),
    out_type=(
        jax.ShapeDtypeStruct((_N_TOKENS,), jnp.float32),
        jax.ShapeDtypeStruct((_N_TOKENS,), jnp.int32),
        jax.ShapeDtypeStruct((_N_TOKENS,), jnp.int32),
    ),
    scratch_types=[
        pltpu.VMEM((_N_EXPERTS, _CHUNK), jnp.float32),
        pltpu.VMEM((_N_EXPERTS, _CHUNK), jnp.float32),
        pltpu.VMEM((_ROWS_PER_W,), jnp.float32),
        pltpu.VMEM((_ROWS_PER_W,), jnp.int32),
        pltpu.VMEM((_ROWS_PER_W,), jnp.int32),
        pltpu.SemaphoreType.DMA,
        pltpu.SemaphoreType.DMA,
    ],
    compiler_params=pltpu.CompilerParams(needs_layout_passes=False),
)(_route_body)


def _materialize_body(p1_ref, i0_ref, i1_ref, gate_ref, idx_ref):
    p1 = p1_ref[...]
    i0 = i0_ref[...]
    i1 = i1_ref[...]
    e = lax.broadcasted_iota(jnp.int32, (_OUT_BLK, _N_EXPERTS), 1)
    i0b = i0[:, None]
    i1b = i1[:, None]
    p1b = p1[:, None]
    gate_ref[...] = jnp.where(
        e == i0b, 1.0 - p1b, jnp.where(e == i1b, p1b, 0.0))
    idx_ref[...] = jnp.concatenate([i0b, i1b], axis=1)


def _materialize(p1, i0, i1):
    return pl.pallas_call(
        _materialize_body,
        grid=(_N_TOKENS // _OUT_BLK,),
        in_specs=[
            pl.BlockSpec((_OUT_BLK,), lambda i: (i,)),
            pl.BlockSpec((_OUT_BLK,), lambda i: (i,)),
            pl.BlockSpec((_OUT_BLK,), lambda i: (i,)),
        ],
        out_specs=[
            pl.BlockSpec((_OUT_BLK, _N_EXPERTS), lambda i: (i, 0)),
            pl.BlockSpec((_OUT_BLK, 2), lambda i: (i, 0)),
        ],
        out_shape=[
            jax.ShapeDtypeStruct((_N_TOKENS, _N_EXPERTS), jnp.float32),
            jax.ShapeDtypeStruct((_N_TOKENS, 2), jnp.int32),
        ],
    )(p1, i0, i1)


@jax.jit
def kernel(x, W, b):
    b2d = jnp.broadcast_to(b[:, None], (_N_EXPERTS, 128))
    lt = _logits_t(x, W, b2d)
    p1, i0, i1 = _route(lt)
    gate, idx = _materialize(p1, i0, i1)
    return (gate, idx)
